# Initial kernel scaffold; baseline (speedup 1.0000x reference)
import jax
import jax.numpy as jnp
from jax.experimental import pallas as pl


def kernel(x, _pos, embedding_table, positional_table):
    raise NotImplementedError("write your pallas kernel here")


# speedup vs baseline: 2.0074x; 2.0074x over previous
import functools

import jax
import jax.numpy as jnp
from jax import lax
from jax.experimental import pallas as pl
from jax.experimental.pallas import tpu as pltpu
from jax.experimental.pallas import tpu_sc as plsc

B = 4096
L = 200
D = 64
N = B * L

NC = 2
NS = 16
NW = NC * NS
R = N // NW
C = 512
SUB = 128
NSUB = C // SUB
NCHUNK = R // C
IDX_ROWS = N // SUB


def _impl(x2d, pos2d, emb, ptab, rowid):
    mesh = plsc.VectorSubcoreMesh(core_axis_name="c", subcore_axis_name="s")

    @functools.partial(
        pl.kernel,
        mesh=mesh,
        compiler_params=pltpu.CompilerParams(use_tc_tiling_on_sc=False),
        out_type=jax.ShapeDtypeStruct((N, D), jnp.float32),
        scratch_types=[
            pltpu.VMEM((NSUB, SUB), jnp.int32),
            pltpu.VMEM((NSUB, SUB), jnp.int32),
            pltpu.VMEM((NSUB, SUB), jnp.int32),
            pltpu.VMEM((C, D), jnp.float32),
            pltpu.VMEM((C, D), jnp.float32),
            pltpu.SemaphoreType.DMA,
            pltpu.SemaphoreType.DMA,
            pltpu.SemaphoreType.DMA,
        ],
    )
    def k(x_hbm, p_hbm, emb_hbm, ptab_hbm, rowid_hbm, out_hbm,
          idx_v, pidx_v, rowid_v, rows_v, pos_v, sem_g, sem_a, sem_o):
        wid = lax.axis_index("s") * NC + lax.axis_index("c")
        pltpu.sync_copy(rowid_hbm, rowid_v)

        def chunk(g, carry):
            irow = wid * (R // SUB) + g * NSUB
            base = wid * R + g * C
            pltpu.sync_copy(x_hbm.at[pl.ds(irow, NSUB)], idx_v)
            pltpu.sync_copy(p_hbm.at[pl.ds(irow, NSUB)], pidx_v)
            hs = [
                pltpu.async_copy(
                    emb_hbm.at[idx_v.at[j]],
                    rows_v.at[pl.ds(j * SUB, SUB)], sem_g)
                for j in range(NSUB)
            ]
            for h in hs:
                h.wait()
            hs = [
                pltpu.async_copy(
                    ptab_hbm.at[pidx_v.at[j]],
                    rows_v.at[pl.ds(j * SUB, SUB)], sem_a, add=True)
                for j in range(NSUB)
            ]
            for h in hs:
                h.wait()
            pltpu.async_copy(rows_v, out_hbm.at[pl.ds(base, C)], sem_o).wait()
            return carry

        lax.fori_loop(0, NCHUNK, chunk, 0)

    return k(x2d, pos2d, emb, ptab, rowid)


def kernel(x, _pos, embedding_table, positional_table):
    x2d = x.reshape(IDX_ROWS, SUB)
    pos2d = _pos.reshape(IDX_ROWS, SUB)
    rowid = jnp.arange(C, dtype=jnp.int32).reshape(NSUB, SUB)
    out = _impl(x2d, pos2d, embedding_table, positional_table, rowid)
    return out.reshape(B, L, D)
